# Initial kernel scaffold; baseline (speedup 1.0000x reference)
import functools

import jax
import jax.numpy as jnp
from jax import lax
from jax.experimental import pallas as pl
from jax.experimental.pallas import tpu as pltpu
from jax.experimental.pallas import tpu_sc as plsc

_VOCAB = 100000
_HIDDEN = 64
_BATCH = 1024
_HIST = 200

_NC = 2
_NS = 16
_NW = _NC * _NS
_BPW = _BATCH // _NW
_HALF = _HIST // 2


def _pool_body(x_hbm, table_hbm, ctx_hbm, idx_v, rows_v, ctx_v, sem):
    wid = lax.axis_index("s") * _NC + lax.axis_index("c")
    base = wid * _BPW
    pltpu.sync_copy(x_hbm.at[pl.ds(base, _BPW)], idx_v)

    def row(r, carry):
        cp0 = pltpu.async_copy(
            table_hbm.at[idx_v.at[r, 0]], rows_v.at[pl.ds(0, _HALF)], sem)
        cp1 = pltpu.async_copy(
            table_hbm.at[idx_v.at[r, 1]], rows_v.at[pl.ds(_HALF, _HALF)], sem)
        cp0.wait()
        cp1.wait()

        def red(t, accs):
            a0, a1, a2, a3 = accs
            a0 = a0 + rows_v[t, pl.ds(0, 16)]
            a1 = a1 + rows_v[t, pl.ds(16, 16)]
            a2 = a2 + rows_v[t, pl.ds(32, 16)]
            a3 = a3 + rows_v[t, pl.ds(48, 16)]
            return (a0, a1, a2, a3)

        z = jnp.zeros((16,), jnp.float32)
        a0, a1, a2, a3 = lax.fori_loop(0, _HIST, red, (z, z, z, z))
        s = jnp.float32(1.0 / _HIST)
        ctx_v[r, pl.ds(0, 16)] = a0 * s
        ctx_v[r, pl.ds(16, 16)] = a1 * s
        ctx_v[r, pl.ds(32, 16)] = a2 * s
        ctx_v[r, pl.ds(48, 16)] = a3 * s
        return carry

    lax.fori_loop(0, _BPW, row, 0)
    pltpu.sync_copy(ctx_v, ctx_hbm.at[pl.ds(base, _BPW)])


@functools.partial(
    pl.kernel,
    out_type=jax.ShapeDtypeStruct((_BATCH, _HIDDEN), jnp.float32),
    mesh=plsc.VectorSubcoreMesh(core_axis_name="c", subcore_axis_name="s"),
    scratch_types=[
        pltpu.VMEM((_BPW, 2, _HALF), jnp.int32),
        pltpu.VMEM((_HIST, _HIDDEN), jnp.float32),
        pltpu.VMEM((_BPW, _HIDDEN), jnp.float32),
        pltpu.SemaphoreType.DMA,
    ],
)
def _pool(x_hbm, table_hbm, ctx_hbm, idx_v, rows_v, ctx_v, sem):
    _pool_body(x_hbm, table_hbm, ctx_hbm, idx_v, rows_v, ctx_v, sem)


_BV = 2048


def _mm_body(ctx_ref, w_ref, b_ref, out_ref):
    out_ref[...] = lax.dot_general(
        ctx_ref[...], w_ref[...],
        dimension_numbers=(((1,), (1,)), ((), ())),
        preferred_element_type=jnp.float32,
    ) + b_ref[...]


def _matmul(context, W, b2):
    return pl.pallas_call(
        _mm_body,
        grid=(pl.cdiv(_VOCAB, _BV),),
        in_specs=[
            pl.BlockSpec((_BATCH, _HIDDEN), lambda j: (0, 0)),
            pl.BlockSpec((_BV, _HIDDEN), lambda j: (j, 0)),
            pl.BlockSpec((1, _BV), lambda j: (0, j)),
        ],
        out_specs=pl.BlockSpec((_BATCH, _BV), lambda j: (0, j)),
        out_shape=jax.ShapeDtypeStruct((_BATCH, _VOCAB), jnp.float32),
        compiler_params=pltpu.CompilerParams(
            dimension_semantics=("arbitrary",)),
    )(context, W, b2)


def kernel(x, table, W, b):
    x3 = x.astype(jnp.int32).reshape(_BATCH, 2, _HALF)
    context = _pool(x3, table)
    return _matmul(context, W, b.reshape(1, _VOCAB))


# speedup vs baseline: 1.3843x; 1.3843x over previous
import functools

import jax
import jax.numpy as jnp
from jax import lax
from jax.experimental import pallas as pl
from jax.experimental.pallas import tpu as pltpu
from jax.experimental.pallas import tpu_sc as plsc

_VOCAB = 100000
_HIDDEN = 64
_BATCH = 1024
_HIST = 200

_NC = 2
_NS = 16
_NW = _NC * _NS
_BPW = _BATCH // _NW
_HALF = _HIST // 2


def _pool_body(x_hbm, table_hbm, ctx_hbm, idx_v, rows_v, ctx_v, sem):
    wid = lax.axis_index("s") * _NC + lax.axis_index("c")
    base = wid * _BPW
    pltpu.sync_copy(x_hbm.at[pl.ds(base, _BPW)], idx_v)

    def row(r, carry):
        cp0 = pltpu.async_copy(
            table_hbm.at[idx_v.at[r, 0]], rows_v.at[pl.ds(0, _HALF)], sem)
        cp1 = pltpu.async_copy(
            table_hbm.at[idx_v.at[r, 1]], rows_v.at[pl.ds(_HALF, _HALF)], sem)
        cp0.wait()
        cp1.wait()

        def red(t, accs):
            a0, a1, a2, a3 = accs
            a0 = a0 + rows_v[t, pl.ds(0, 16)]
            a1 = a1 + rows_v[t, pl.ds(16, 16)]
            a2 = a2 + rows_v[t, pl.ds(32, 16)]
            a3 = a3 + rows_v[t, pl.ds(48, 16)]
            return (a0, a1, a2, a3)

        z = jnp.zeros((16,), jnp.float32)
        a0, a1, a2, a3 = lax.fori_loop(0, _HIST, red, (z, z, z, z))
        s = jnp.float32(1.0 / _HIST)
        ctx_v[r, pl.ds(0, 16)] = a0 * s
        ctx_v[r, pl.ds(16, 16)] = a1 * s
        ctx_v[r, pl.ds(32, 16)] = a2 * s
        ctx_v[r, pl.ds(48, 16)] = a3 * s
        return carry

    lax.fori_loop(0, _BPW, row, 0)
    pltpu.sync_copy(ctx_v, ctx_hbm.at[pl.ds(base, _BPW)])


@functools.partial(
    pl.kernel,
    out_type=jax.ShapeDtypeStruct((_BATCH, _HIDDEN), jnp.float32),
    mesh=plsc.VectorSubcoreMesh(core_axis_name="c", subcore_axis_name="s"),
    scratch_types=[
        pltpu.VMEM((_BPW, 2, _HALF), jnp.int32),
        pltpu.VMEM((_HIST, _HIDDEN), jnp.float32),
        pltpu.VMEM((_BPW, _HIDDEN), jnp.float32),
        pltpu.SemaphoreType.DMA,
    ],
    compiler_params=pltpu.CompilerParams(use_tc_tiling_on_sc=False),
)
def _pool(x_hbm, table_hbm, ctx_hbm, idx_v, rows_v, ctx_v, sem):
    _pool_body(x_hbm, table_hbm, ctx_hbm, idx_v, rows_v, ctx_v, sem)


_BV = 2048


def _mm_body(ctx_ref, w_ref, b_ref, out_ref):
    out_ref[...] = lax.dot_general(
        ctx_ref[...], w_ref[...],
        dimension_numbers=(((1,), (1,)), ((), ())),
        preferred_element_type=jnp.float32,
    ) + b_ref[...]


def _matmul(context, W, b2):
    return pl.pallas_call(
        _mm_body,
        grid=(pl.cdiv(_VOCAB, _BV),),
        in_specs=[
            pl.BlockSpec((_BATCH, _HIDDEN), lambda j: (0, 0)),
            pl.BlockSpec((_BV, _HIDDEN), lambda j: (j, 0)),
            pl.BlockSpec((1, _BV), lambda j: (0, j)),
        ],
        out_specs=pl.BlockSpec((_BATCH, _BV), lambda j: (0, j)),
        out_shape=jax.ShapeDtypeStruct((_BATCH, _VOCAB), jnp.float32),
        compiler_params=pltpu.CompilerParams(
            dimension_semantics=("arbitrary",)),
    )(context, W, b2)


def kernel(x, table, W, b):
    x3 = x.astype(jnp.int32).reshape(_BATCH, 2, _HALF)
    context = _pool(x3, table)
    return _matmul(context, W, b.reshape(1, _VOCAB))
